# Initial kernel scaffold; baseline (speedup 1.0000x reference)
import jax
import jax.numpy as jnp
from jax.experimental import pallas as pl


def kernel(cat, col, fab, store, cat_table, col_table, fab_table, store_table):
    raise NotImplementedError("write your pallas kernel here")


# speedup vs baseline: 2.2641x; 2.2641x over previous
import functools

import jax
import jax.numpy as jnp
from jax import lax
from jax.experimental import pallas as pl
from jax.experimental.pallas import tpu as pltpu
from jax.experimental.pallas import tpu_sc as plsc

B = 16384
D = 32
NUM_TABLES = 4

_info = plsc.get_sparse_core_info()
NC = _info.num_cores
NS = _info.num_subcores
NW = NC * NS
BPW = B // NW


@functools.partial(
    pl.kernel,
    out_type=jax.ShapeDtypeStruct((B, NUM_TABLES, D), jnp.float32),
    mesh=plsc.VectorSubcoreMesh(core_axis_name="c", subcore_axis_name="s"),
    compiler_params=pltpu.CompilerParams(use_tc_tiling_on_sc=False),
    scratch_types=(
        [pltpu.VMEM((BPW,), jnp.int32) for _ in range(NUM_TABLES)]
        + [pltpu.VMEM((BPW, D), jnp.float32) for _ in range(NUM_TABLES)]
        + [pltpu.SemaphoreType.DMA for _ in range(NUM_TABLES)]
    ),
)
def _encode(cat_h, col_h, fab_h, store_h,
            cat_t, col_t, fab_t, store_t,
            out_h,
            i0, i1, i2, i3, r0, r1, r2, r3, s0, s1, s2, s3):
    wid = lax.axis_index("s") * NC + lax.axis_index("c")
    base = wid * BPW
    idx_refs = (i0, i1, i2, i3)
    row_refs = (r0, r1, r2, r3)
    sems = (s0, s1, s2, s3)
    idx_srcs = (cat_h, col_h, fab_h, store_h)
    tables = (cat_t, col_t, fab_t, store_t)

    copies = []
    for t in range(NUM_TABLES):
        pltpu.sync_copy(idx_srcs[t].at[pl.ds(base, BPW)], idx_refs[t])
        copies.append(pltpu.async_copy(tables[t].at[idx_refs[t]], row_refs[t], sems[t]))
    for t in range(NUM_TABLES):
        copies[t].wait()
        pltpu.sync_copy(row_refs[t], out_h.at[pl.ds(base, BPW), t])


def kernel(cat, col, fab, store, cat_table, col_table, fab_table, store_table):
    return _encode(cat, col, fab, store,
                   cat_table, col_table, fab_table, store_table)


# speedup vs baseline: 2.2927x; 1.0126x over previous
import functools

import jax
import jax.numpy as jnp
from jax import lax
from jax.experimental import pallas as pl
from jax.experimental.pallas import tpu as pltpu
from jax.experimental.pallas import tpu_sc as plsc

B = 16384
D = 32
NUM_TABLES = 4

_info = plsc.get_sparse_core_info()
NC = _info.num_cores
NS = _info.num_subcores
NW = NC * NS
BPW = B // NW


@functools.partial(
    pl.kernel,
    out_type=jax.ShapeDtypeStruct((B, NUM_TABLES, D), jnp.float32),
    mesh=plsc.VectorSubcoreMesh(core_axis_name="c", subcore_axis_name="s"),
    compiler_params=pltpu.CompilerParams(use_tc_tiling_on_sc=False),
    scratch_types=(
        [pltpu.VMEM((BPW,), jnp.int32) for _ in range(NUM_TABLES)]
        + [pltpu.VMEM((BPW, D), jnp.float32) for _ in range(NUM_TABLES)]
        + [pltpu.SemaphoreType.DMA for _ in range(3 * NUM_TABLES)]
    ),
)
def _encode(cat_h, col_h, fab_h, store_h,
            cat_t, col_t, fab_t, store_t,
            out_h,
            i0, i1, i2, i3, r0, r1, r2, r3, *sems):
    wid = lax.axis_index("s") * NC + lax.axis_index("c")
    base = wid * BPW
    idx_refs = (i0, i1, i2, i3)
    row_refs = (r0, r1, r2, r3)
    idx_srcs = (cat_h, col_h, fab_h, store_h)
    tables = (cat_t, col_t, fab_t, store_t)

    idx_cp = [pltpu.async_copy(idx_srcs[t].at[pl.ds(base, BPW)], idx_refs[t],
                               sems[t])
              for t in range(NUM_TABLES)]
    row_cp = []
    for t in range(NUM_TABLES):
        idx_cp[t].wait()
        row_cp.append(pltpu.async_copy(tables[t].at[idx_refs[t]], row_refs[t],
                                       sems[NUM_TABLES + t]))
    out_cp = []
    for t in range(NUM_TABLES):
        row_cp[t].wait()
        out_cp.append(pltpu.async_copy(row_refs[t],
                                       out_h.at[pl.ds(base, BPW), t],
                                       sems[2 * NUM_TABLES + t]))
    for t in range(NUM_TABLES):
        out_cp[t].wait()


def kernel(cat, col, fab, store, cat_table, col_table, fab_table, store_table):
    return _encode(cat, col, fab, store,
                   cat_table, col_table, fab_table, store_table)


# speedup vs baseline: 5.1594x; 2.2504x over previous
import functools

import jax
import jax.numpy as jnp
from jax import lax
from jax.experimental import pallas as pl
from jax.experimental.pallas import tpu as pltpu
from jax.experimental.pallas import tpu_sc as plsc

B = 16384
D = 32
NUM_TABLES = 4
V_SMALL = 1000
V_STORE = 100000
CH = 4096
NCH = B // CH
L = 16

_info = plsc.get_sparse_core_info()
NC = _info.num_cores
NS = _info.num_subcores
NW = NC * NS


@functools.partial(
    pl.kernel,
    out_type=jax.ShapeDtypeStruct((NUM_TABLES, D, B), jnp.float32),
    mesh=plsc.VectorSubcoreMesh(core_axis_name="c", subcore_axis_name="s"),
    compiler_params=pltpu.CompilerParams(use_tc_tiling_on_sc=True,
                                         needs_layout_passes=False),
    scratch_types=(
        [pltpu.VMEM((V_STORE,), jnp.float32)]
        + [pltpu.VMEM((V_SMALL,), jnp.float32) for _ in range(3)]
        + [pltpu.VMEM((CH,), jnp.int32) for _ in range(2)]
        + [pltpu.VMEM((CH,), jnp.float32) for _ in range(2)]
        + [pltpu.SemaphoreType.DMA for _ in range(10)]
    ),
)
def _encode(cat_h, col_h, fab_h, store_h,
            cat_t, col_t, fab_t, store_t,
            out_h,
            store_row, row0, row1, row2,
            idx0, idx1, ob0, ob1,
            *sems):
    k = lax.axis_index("s") * NC + lax.axis_index("c")
    idx_srcs = (cat_h, col_h, fab_h, store_h)
    rows = (row0, row1, row2, store_row)
    idx_bufs = (idx0, idx1)
    out_bufs = (ob0, ob1)
    row_sems = sems[0:4]
    idx_sems = sems[4:6]
    out_sems = sems[6:8]
    gather_sems = sems[8:10]

    row_cp = [
        pltpu.async_copy(cat_t.at[k], row0, row_sems[0]),
        pltpu.async_copy(col_t.at[k], row1, row_sems[1]),
        pltpu.async_copy(fab_t.at[k], row2, row_sems[2]),
        pltpu.async_copy(store_t.at[k], store_row, row_sems[3]),
    ]

    stages = [(t, c) for t in range(NUM_TABLES) for c in range(NCH)]
    idx_cp = {}
    out_cp = {}
    t0, c0 = stages[0]
    idx_cp[0] = pltpu.async_copy(
        idx_srcs[t0].at[pl.ds(c0 * CH, CH)], idx_bufs[0], idx_sems[0])

    for s, (t, c) in enumerate(stages):
        if s + 1 < len(stages):
            tn, cn = stages[s + 1]
            idx_cp[s + 1] = pltpu.async_copy(
                idx_srcs[tn].at[pl.ds(cn * CH, CH)],
                idx_bufs[(s + 1) % 2], idx_sems[(s + 1) % 2])
        if c == 0:
            row_cp[t].wait()
        idx_cp.pop(s).wait()
        if s >= 2:
            out_cp.pop(s - 2).wait()
        ib = idx_bufs[s % 2]
        ob = out_bufs[s % 2]
        row = rows[t]

        def body(i, _):
            iv = ib[pl.ds(i * L, L)]
            ob[pl.ds(i * L, L)] = plsc.load_gather(row, [iv])
            return 0

        lax.fori_loop(0, CH // L, body, 0)
        out_cp[s] = pltpu.async_copy(
            ob, out_h.at[t, k, pl.ds(c * CH, CH)], out_sems[s % 2])

    for s in sorted(out_cp):
        out_cp[s].wait()


def kernel(cat, col, fab, store, cat_table, col_table, fab_table, store_table):
    out_phys = _encode(cat, col, fab, store,
                       cat_table.T, col_table.T, fab_table.T, store_table.T)
    return jnp.transpose(out_phys, (2, 0, 1))


# speedup vs baseline: 5.4339x; 1.0532x over previous
import functools

import jax
import jax.numpy as jnp
from jax import lax
from jax.experimental import pallas as pl
from jax.experimental.pallas import tpu as pltpu
from jax.experimental.pallas import tpu_sc as plsc

B = 16384
D = 32
NUM_TABLES = 4
V_SMALL = 1000
V_STORE = 100000
CH = 4096
NCH = B // CH
L = 16
UNROLL = 8

_info = plsc.get_sparse_core_info()
NC = _info.num_cores
NS = _info.num_subcores
NW = NC * NS


@functools.partial(
    pl.kernel,
    out_type=jax.ShapeDtypeStruct((NUM_TABLES, D, B), jnp.float32),
    mesh=plsc.VectorSubcoreMesh(core_axis_name="c", subcore_axis_name="s"),
    compiler_params=pltpu.CompilerParams(use_tc_tiling_on_sc=True,
                                         needs_layout_passes=False),
    scratch_types=(
        [pltpu.VMEM((V_STORE,), jnp.float32)]
        + [pltpu.VMEM((V_SMALL,), jnp.float32) for _ in range(3)]
        + [pltpu.VMEM((CH,), jnp.int32) for _ in range(2)]
        + [pltpu.VMEM((CH,), jnp.float32) for _ in range(2)]
        + [pltpu.SemaphoreType.DMA for _ in range(10)]
    ),
)
def _encode(cat_h, col_h, fab_h, store_h,
            cat_t, col_t, fab_t, store_t,
            out_h,
            store_row, row0, row1, row2,
            idx0, idx1, ob0, ob1,
            *sems):
    k = lax.axis_index("s") * NC + lax.axis_index("c")
    idx_srcs = (cat_h, col_h, fab_h, store_h)
    rows = (row0, row1, row2, store_row)
    idx_bufs = (idx0, idx1)
    out_bufs = (ob0, ob1)
    row_sems = sems[0:4]
    idx_sems = sems[4:6]
    out_sems = sems[6:8]
    gather_sems = sems[8:10]

    row_cp = [
        pltpu.async_copy(cat_t.at[k], row0, row_sems[0]),
        pltpu.async_copy(col_t.at[k], row1, row_sems[1]),
        pltpu.async_copy(fab_t.at[k], row2, row_sems[2]),
        pltpu.async_copy(store_t.at[k], store_row, row_sems[3]),
    ]

    stages = [(t, c) for t in range(NUM_TABLES) for c in range(NCH)]
    idx_cp = {}
    out_cp = {}
    t0, c0 = stages[0]
    idx_cp[0] = pltpu.async_copy(
        idx_srcs[t0].at[pl.ds(c0 * CH, CH)], idx_bufs[0], idx_sems[0])

    for s, (t, c) in enumerate(stages):
        if s + 1 < len(stages):
            tn, cn = stages[s + 1]
            idx_cp[s + 1] = pltpu.async_copy(
                idx_srcs[tn].at[pl.ds(cn * CH, CH)],
                idx_bufs[(s + 1) % 2], idx_sems[(s + 1) % 2])
        if c == 0:
            row_cp[t].wait()
        idx_cp.pop(s).wait()
        if s >= 2:
            out_cp.pop(s - 2).wait()
        ib = idx_bufs[s % 2]
        ob = out_bufs[s % 2]
        row = rows[t]

        def body(i, _):
            base = i * (L * UNROLL)
            for u in range(UNROLL):
                iv = ib[pl.ds(base + u * L, L)]
                ob[pl.ds(base + u * L, L)] = plsc.load_gather(row, [iv])
            return 0

        lax.fori_loop(0, CH // (L * UNROLL), body, 0)
        out_cp[s] = pltpu.async_copy(
            ob, out_h.at[t, k, pl.ds(c * CH, CH)], out_sems[s % 2])

    for s in sorted(out_cp):
        out_cp[s].wait()


def kernel(cat, col, fab, store, cat_table, col_table, fab_table, store_table):
    out_phys = _encode(cat, col, fab, store,
                       cat_table.T, col_table.T, fab_table.T, store_table.T)
    return jnp.transpose(out_phys, (2, 0, 1))
